# baseline (speedup 1.0000x reference)
import functools

import jax
import jax.numpy as jnp
from jax import lax
from jax.experimental import pallas as pl
from jax.experimental.pallas import tpu as pltpu
from jax.experimental.pallas import tpu_sc as plsc

NC, NS, L = 2, 16, 16
NW = NC * NS
ROWS = 64 * 32
ROW_LEN = 32768
NIDX = 1024
ROWS_PER_W = ROWS // NW

_mesh = plsc.VectorSubcoreMesh(
    core_axis_name="c", subcore_axis_name="s", num_cores=NC, num_subcores=NS
)


@functools.partial(
    pl.kernel,
    out_type=jax.ShapeDtypeStruct((ROWS, NIDX), jnp.float32),
    mesh=_mesh,
    compiler_params=pltpu.CompilerParams(needs_layout_passes=False),
    scratch_types=[
        pltpu.VMEM((ROW_LEN,), jnp.float32),
        pltpu.VMEM((NIDX,), jnp.int32),
        pltpu.VMEM((NIDX,), jnp.float32),
    ],
)
def _sc_gather(t_hbm, i_hbm, o_hbm, row_v, idx_v, out_v):
    wid = lax.axis_index("s") * NC + lax.axis_index("c")
    base = wid * ROWS_PER_W

    def per_row(r, _):
        row = base + r
        pltpu.sync_copy(t_hbm.at[row], row_v)
        pltpu.sync_copy(i_hbm.at[row], idx_v)

        def gather16(i, _):
            iv = idx_v[pl.ds(i * L, L)]
            out_v[pl.ds(i * L, L)] = plsc.load_gather(row_v, [iv])
            return 0

        lax.fori_loop(0, NIDX // L, gather16, 0)
        pltpu.sync_copy(out_v, o_hbm.at[row])
        return 0

    lax.fori_loop(0, ROWS_PER_W, per_row, 0)


def kernel(tensor, indices):
    t = tensor.reshape(ROWS, ROW_LEN)
    ix = indices.reshape(ROWS, NIDX)
    out = _sc_gather(t, ix)
    return out.reshape(indices.shape)


# speedup vs baseline: 1.5453x; 1.5453x over previous
import functools

import jax
import jax.numpy as jnp
from jax import lax
from jax.experimental import pallas as pl
from jax.experimental.pallas import tpu as pltpu
from jax.experimental.pallas import tpu_sc as plsc

NC, NS, L = 2, 16, 16
NW = NC * NS
ROWS = 64 * 32
ROW_LEN = 32768
NIDX = 1024
ROWS_PER_W = ROWS // NW
NPAIR = ROWS_PER_W // 2

_mesh = plsc.VectorSubcoreMesh(
    core_axis_name="c", subcore_axis_name="s", num_cores=NC, num_subcores=NS
)


@functools.partial(
    pl.kernel,
    out_type=jax.ShapeDtypeStruct((ROWS, NIDX), jnp.float32),
    mesh=_mesh,
    compiler_params=pltpu.CompilerParams(needs_layout_passes=False),
    scratch_types=[
        pltpu.VMEM((ROW_LEN,), jnp.float32),
        pltpu.VMEM((ROW_LEN,), jnp.float32),
        pltpu.VMEM((NIDX,), jnp.int32),
        pltpu.VMEM((NIDX,), jnp.int32),
        pltpu.VMEM((NIDX,), jnp.float32),
        pltpu.VMEM((NIDX,), jnp.float32),
        pltpu.SemaphoreType.DMA,
        pltpu.SemaphoreType.DMA,
        pltpu.SemaphoreType.DMA,
        pltpu.SemaphoreType.DMA,
    ],
)
def _sc_gather(t_hbm, i_hbm, o_hbm, row0_v, row1_v, idx0_v, idx1_v,
               out0_v, out1_v, rsem0, rsem1, osem0, osem1):
    wid = lax.axis_index("s") * NC + lax.axis_index("c")
    base = wid * ROWS_PER_W
    rows_v = (row0_v, row1_v)
    idxs_v = (idx0_v, idx1_v)
    outs_v = (out0_v, out1_v)
    rsems = (rsem0, rsem1)
    osems = (osem0, osem1)

    def start_in(row, p):
        pltpu.async_copy(t_hbm.at[row], rows_v[p], rsems[p])
        pltpu.async_copy(i_hbm.at[row], idxs_v[p], rsems[p])

    def wait_in(row, p):
        pltpu.make_async_copy(t_hbm.at[row], rows_v[p], rsems[p]).wait()
        pltpu.make_async_copy(i_hbm.at[row], idxs_v[p], rsems[p]).wait()

    def wait_out(p):
        pltpu.make_async_copy(outs_v[p], o_hbm.at[base], osems[p]).wait()

    def do_row(row, p):
        def gather16(i, _):
            iv = idxs_v[p][pl.ds(i * L, L)]
            outs_v[p][pl.ds(i * L, L)] = plsc.load_gather(rows_v[p], [iv])
            return 0

        lax.fori_loop(0, NIDX // L, gather16, 0, unroll=4)
        pltpu.async_copy(outs_v[p], o_hbm.at[row], osems[p])

    start_in(base, 0)

    def pair(g, _):
        r0 = base + 2 * g
        start_in(r0 + 1, 1)
        wait_in(r0, 0)
        @pl.when(g > 0)
        def _():
            wait_out(0)
        do_row(r0, 0)
        @pl.when(g < NPAIR - 1)
        def _():
            start_in(r0 + 2, 0)
        wait_in(r0 + 1, 1)
        @pl.when(g > 0)
        def _():
            wait_out(1)
        do_row(r0 + 1, 1)
        return 0

    lax.fori_loop(0, NPAIR, pair, 0)
    wait_out(0)
    wait_out(1)


def kernel(tensor, indices):
    t = tensor.reshape(ROWS, ROW_LEN)
    ix = indices.reshape(ROWS, NIDX)
    out = _sc_gather(t, ix)
    return out.reshape(indices.shape)


# speedup vs baseline: 1.5484x; 1.0020x over previous
import functools

import jax
import jax.numpy as jnp
from jax import lax
from jax.experimental import pallas as pl
from jax.experimental.pallas import tpu as pltpu
from jax.experimental.pallas import tpu_sc as plsc

NC, NS, L = 2, 16, 16
NW = NC * NS
ROWS = 64 * 32
ROW_LEN = 32768
NIDX = 1024
ROWS_PER_W = ROWS // NW
NPAIR = ROWS_PER_W // 2

_mesh = plsc.VectorSubcoreMesh(
    core_axis_name="c", subcore_axis_name="s", num_cores=NC, num_subcores=NS
)


@functools.partial(
    pl.kernel,
    out_type=jax.ShapeDtypeStruct((ROWS, NIDX), jnp.float32),
    mesh=_mesh,
    compiler_params=pltpu.CompilerParams(needs_layout_passes=False),
    scratch_types=[
        pltpu.VMEM((ROW_LEN,), jnp.float32),
        pltpu.VMEM((ROW_LEN,), jnp.float32),
        pltpu.VMEM((NIDX,), jnp.int32),
        pltpu.VMEM((NIDX,), jnp.int32),
        pltpu.VMEM((NIDX,), jnp.float32),
        pltpu.VMEM((NIDX,), jnp.float32),
        pltpu.SemaphoreType.DMA,
        pltpu.SemaphoreType.DMA,
        pltpu.SemaphoreType.DMA,
        pltpu.SemaphoreType.DMA,
    ],
)
def _sc_gather(t_hbm, i_hbm, o_hbm, row0_v, row1_v, idx0_v, idx1_v,
               out0_v, out1_v, rsem0, rsem1, osem0, osem1):
    wid = lax.axis_index("s") * NC + lax.axis_index("c")
    base = wid * ROWS_PER_W
    rows_v = (row0_v, row1_v)
    idxs_v = (idx0_v, idx1_v)
    outs_v = (out0_v, out1_v)
    rsems = (rsem0, rsem1)
    osems = (osem0, osem1)

    NSPLIT = 4
    CH = ROW_LEN // NSPLIT

    def start_in(row, p):
        for c in range(NSPLIT):
            sl = pl.ds(c * CH, CH)
            pltpu.async_copy(t_hbm.at[row, sl], rows_v[p].at[sl], rsems[p])
        pltpu.async_copy(i_hbm.at[row], idxs_v[p], rsems[p])

    def wait_in(row, p):
        for c in range(NSPLIT):
            sl = pl.ds(c * CH, CH)
            pltpu.make_async_copy(t_hbm.at[row, sl], rows_v[p].at[sl],
                                  rsems[p]).wait()
        pltpu.make_async_copy(i_hbm.at[row], idxs_v[p], rsems[p]).wait()

    def wait_out(p):
        pltpu.make_async_copy(outs_v[p], o_hbm.at[base], osems[p]).wait()

    def do_row(row, p):
        def gather16(i, _):
            iv = idxs_v[p][pl.ds(i * L, L)]
            outs_v[p][pl.ds(i * L, L)] = plsc.load_gather(rows_v[p], [iv])
            return 0

        lax.fori_loop(0, NIDX // L, gather16, 0, unroll=4)
        pltpu.async_copy(outs_v[p], o_hbm.at[row], osems[p])

    start_in(base, 0)

    def pair(g, _):
        r0 = base + 2 * g
        start_in(r0 + 1, 1)
        wait_in(r0, 0)
        @pl.when(g > 0)
        def _():
            wait_out(0)
        do_row(r0, 0)
        @pl.when(g < NPAIR - 1)
        def _():
            start_in(r0 + 2, 0)
        wait_in(r0 + 1, 1)
        @pl.when(g > 0)
        def _():
            wait_out(1)
        do_row(r0 + 1, 1)
        return 0

    lax.fori_loop(0, NPAIR, pair, 0)
    wait_out(0)
    wait_out(1)


def kernel(tensor, indices):
    t = tensor.reshape(ROWS, ROW_LEN)
    ix = indices.reshape(ROWS, NIDX)
    out = _sc_gather(t, ix)
    return out.reshape(indices.shape)
